# baseline (speedup 1.0000x reference)
import functools

import jax
import jax.numpy as jnp
from jax import lax
from jax.experimental import pallas as pl
from jax.experimental.pallas import tpu as pltpu
from jax.experimental.pallas import tpu_sc as plsc

NUM_USERS = 1000000
NUM_ITEMS = 1000000
EMBED_DIM = 64
BATCH = 16384

_NC = 2
_NS = 16
_NW = _NC * _NS
_BPW = BATCH // _NW
_L = 16


def _mf_kernel(uidx_hbm, iidx_hbm, utab_hbm, itab_hbm, ubias_hbm, ibias_hbm,
               gbias_hbm, out_hbm,
               uidx_v, iidx_v, urows_v, irows_v, ub_v, ib_v, gb_v, dots_v,
               out_v, sem0, sem1, sem2, sem3):
    wid = lax.axis_index("s") * _NC + lax.axis_index("c")
    base = wid * _BPW

    pltpu.sync_copy(uidx_hbm.at[pl.ds(base, _BPW)], uidx_v)
    pltpu.sync_copy(iidx_hbm.at[pl.ds(base, _BPW)], iidx_v)
    pltpu.sync_copy(gbias_hbm, gb_v)

    def clamp_body(j, _):
        sl = pl.ds(j * _L, _L)
        uidx_v[sl] = jnp.clip(uidx_v[sl], 0, NUM_USERS - 1)
        iidx_v[sl] = jnp.clip(iidx_v[sl], 0, NUM_ITEMS - 1)
        return _
    lax.fori_loop(0, _BPW // _L, clamp_body, 0, unroll=4)

    cp0 = pltpu.async_copy(utab_hbm.at[uidx_v], urows_v, sem0)
    cp1 = pltpu.async_copy(itab_hbm.at[iidx_v], irows_v, sem1)
    cp2 = pltpu.async_copy(ubias_hbm.at[uidx_v], ub_v, sem2)
    cp3 = pltpu.async_copy(ibias_hbm.at[iidx_v], ib_v, sem3)
    cp0.wait()
    cp1.wait()
    cp2.wait()
    cp3.wait()

    lane = lax.iota(jnp.int32, _L)
    last_lane = lane == (_L - 1)

    def dot_body(g, _):
        rbase = g * _L
        for r in range(_L):
            row = rbase + r
            p = urows_v[row, pl.ds(0, _L)] * irows_v[row, pl.ds(0, _L)]
            p = p + urows_v[row, pl.ds(_L, _L)] * irows_v[row, pl.ds(_L, _L)]
            p = p + urows_v[row, pl.ds(2 * _L, _L)] * irows_v[row, pl.ds(2 * _L, _L)]
            p = p + urows_v[row, pl.ds(3 * _L, _L)] * irows_v[row, pl.ds(3 * _L, _L)]
            c = plsc.cumsum(p)
            plsc.store_scatter(dots_v, [jnp.full((_L,), row, jnp.int32)], c,
                               mask=last_lane)
        return _
    lax.fori_loop(0, _BPW // _L, dot_body, 0)

    gv = gb_v[pl.ds(0, _L)]

    def epi_body(j, _):
        sl = pl.ds(j * _L, _L)
        pred = dots_v[sl] + ub_v[sl] + ib_v[sl] + gv
        out_v[sl] = 1.0 / (1.0 + jnp.exp(-pred))
        return _
    lax.fori_loop(0, _BPW // _L, epi_body, 0, unroll=4)

    pltpu.sync_copy(out_v, out_hbm.at[pl.ds(base, _BPW)])


@jax.jit
def _run(user_indices, item_indices, user_table, item_table, user_bias,
         item_bias, global_bias):
    mesh = plsc.VectorSubcoreMesh(core_axis_name="c", subcore_axis_name="s")
    k = functools.partial(
        pl.kernel,
        mesh=mesh,
        compiler_params=pltpu.CompilerParams(needs_layout_passes=False,
                                             use_tc_tiling_on_sc=False),
        out_type=jax.ShapeDtypeStruct((BATCH,), jnp.float32),
        scratch_types=[
            pltpu.VMEM((_BPW,), jnp.int32),
            pltpu.VMEM((_BPW,), jnp.int32),
            pltpu.VMEM((_BPW, EMBED_DIM), jnp.float32),
            pltpu.VMEM((_BPW, EMBED_DIM), jnp.float32),
            pltpu.VMEM((_BPW,), jnp.float32),
            pltpu.VMEM((_BPW,), jnp.float32),
            pltpu.VMEM((_L,), jnp.float32),
            pltpu.VMEM((_BPW,), jnp.float32),
            pltpu.VMEM((_BPW,), jnp.float32),
            pltpu.SemaphoreType.DMA,
            pltpu.SemaphoreType.DMA,
            pltpu.SemaphoreType.DMA,
            pltpu.SemaphoreType.DMA,
        ],
    )(_mf_kernel)
    return k(user_indices, item_indices, user_table, item_table,
             user_bias.reshape(NUM_USERS), item_bias.reshape(NUM_ITEMS),
             jnp.broadcast_to(global_bias, (_L,)))


def kernel(user_indices, item_indices, user_table, item_table, user_bias,
           item_bias, global_bias):
    return _run(user_indices, item_indices, user_table, item_table,
                user_bias, item_bias, global_bias)
